# baseline (speedup 1.0000x reference)
import jax
import jax.numpy as jnp
from jax.experimental import pallas as pl
from jax.experimental.pallas import tpu as pltpu


def _pos_kernel(colT_ref, rowT_ref, out_ref, scratch, sem):
    d, h, w = scratch.shape[0] // 2, scratch.shape[1], scratch.shape[2]
    b = out_ref.shape[0]
    scratch[:d] = jnp.broadcast_to(colT_ref[...][:, None, :], (d, h, w))
    scratch[d:] = jnp.broadcast_to(rowT_ref[...][:, :, None], (d, h, w))
    copies = [pltpu.make_async_copy(scratch, out_ref.at[i], sem)
              for i in range(b)]
    for c in copies:
        c.start()
    for c in copies:
        c.wait()


def kernel(x, row_embed, col_embed):
    b = x.shape[0]
    h, w = x.shape[-2], x.shape[-1]
    d = row_embed.shape[1]
    colT = col_embed[:w].T
    rowT = row_embed[:h].T
    return pl.pallas_call(
        _pos_kernel,
        in_specs=[
            pl.BlockSpec(memory_space=pltpu.MemorySpace.VMEM),
            pl.BlockSpec(memory_space=pltpu.MemorySpace.VMEM),
        ],
        out_specs=pl.BlockSpec(memory_space=pl.ANY),
        out_shape=jax.ShapeDtypeStruct((b, 2 * d, h, w), jnp.float32),
        scratch_shapes=[
            pltpu.VMEM((2 * d, h, w), jnp.float32),
            pltpu.SemaphoreType.DMA,
        ],
    )(colT, rowT)


# speedup vs baseline: 2.5985x; 2.5985x over previous
import jax
import jax.numpy as jnp
from jax.experimental import pallas as pl
from jax.experimental.pallas import tpu as pltpu


_NSEM = 8
_NCHUNK = 4


def _pos_kernel(col_ref, row_ref, s_col_ref, s_row_ref, out_ref, scratch, sems):
    d, hw = scratch.shape[0] // 2, scratch.shape[1]
    b = out_ref.shape[0]
    dn = (((0,), (0,)), ((), ()))
    scratch[:d, :] = jax.lax.dot_general(
        col_ref[...], s_col_ref[...], dn,
        preferred_element_type=jnp.float32,
        precision=jax.lax.Precision.HIGHEST)
    scratch[d:, :] = jax.lax.dot_general(
        row_ref[...], s_row_ref[...], dn,
        preferred_element_type=jnp.float32,
        precision=jax.lax.Precision.HIGHEST)
    cs = (2 * d) // _NCHUNK
    copies = []
    q = 0
    for i in range(b):
        for c in range(_NCHUNK):
            copies.append(pltpu.make_async_copy(
                scratch.at[pl.ds(c * cs, cs), :],
                out_ref.at[i, pl.ds(c * cs, cs), :],
                sems.at[q % _NSEM]))
            q += 1
    for cp in copies:
        cp.start()
    for cp in copies:
        cp.wait()


def kernel(x, row_embed, col_embed):
    b = x.shape[0]
    h, w = x.shape[-2], x.shape[-1]
    d = row_embed.shape[1]
    hw = h * w
    k = jnp.arange(hw, dtype=jnp.int32)
    s_col = (k[None, :] % w == jnp.arange(w, dtype=jnp.int32)[:, None]
             ).astype(jnp.float32)
    s_row = (k[None, :] // w == jnp.arange(h, dtype=jnp.int32)[:, None]
             ).astype(jnp.float32)
    out = pl.pallas_call(
        _pos_kernel,
        in_specs=[
            pl.BlockSpec(memory_space=pltpu.MemorySpace.VMEM),
            pl.BlockSpec(memory_space=pltpu.MemorySpace.VMEM),
            pl.BlockSpec(memory_space=pltpu.MemorySpace.VMEM),
            pl.BlockSpec(memory_space=pltpu.MemorySpace.VMEM),
        ],
        out_specs=pl.BlockSpec(memory_space=pl.ANY),
        out_shape=jax.ShapeDtypeStruct((b, 2 * d, hw), jnp.float32),
        scratch_shapes=[
            pltpu.VMEM((2 * d, hw), jnp.float32),
            pltpu.SemaphoreType.DMA((_NSEM,)),
        ],
    )(col_embed[:w], row_embed[:h], s_col, s_row)
    return out.reshape(b, 2 * d, h, w)


# speedup vs baseline: 8.6214x; 3.3178x over previous
import jax
import jax.numpy as jnp
from jax.experimental import pallas as pl
from jax.experimental.pallas import tpu as pltpu


def _pos_kernel(col_ref, row_ref, out_ref, scratch, sems):
    h, w, d2 = scratch.shape
    d = d2 // 2
    b = out_ref.shape[0]
    scratch[:, :, :d] = jnp.broadcast_to(col_ref[...][None, :, :], (h, w, d))
    scratch[:, :, d:] = jnp.broadcast_to(row_ref[...][:, None, :], (h, w, d))
    copies = [pltpu.make_async_copy(scratch, out_ref.at[i],
                                    sems.at[i % sems.shape[0]])
              for i in range(b)]
    for c in copies:
        c.start()
    for c in copies:
        c.wait()


def kernel(x, row_embed, col_embed):
    b = x.shape[0]
    h, w = x.shape[-2], x.shape[-1]
    d = row_embed.shape[1]
    out = pl.pallas_call(
        _pos_kernel,
        in_specs=[
            pl.BlockSpec(memory_space=pltpu.MemorySpace.VMEM),
            pl.BlockSpec(memory_space=pltpu.MemorySpace.VMEM),
        ],
        out_specs=pl.BlockSpec(memory_space=pl.ANY),
        out_shape=jax.ShapeDtypeStruct((b, h, w, 2 * d), jnp.float32),
        scratch_shapes=[
            pltpu.VMEM((h, w, 2 * d), jnp.float32),
            pltpu.SemaphoreType.DMA((8,)),
        ],
    )(col_embed[:w], row_embed[:h])
    return jnp.transpose(out, (0, 3, 1, 2))
